# baseline (speedup 1.0000x reference)
import functools
import math

import jax
import jax.numpy as jnp
from jax import lax
from jax.experimental import pallas as pl
from jax.experimental.pallas import tpu as pltpu
from jax.experimental.pallas import tpu_sc as plsc

D = 64
IDX_CHUNK = 128
NBUF = 2


def _sc_gather_pairs(user_pairs, item_pairs, uhi2, ihi2, n_workers, chunks):
    B = n_workers * chunks * IDX_CHUNK

    mesh = plsc.VectorSubcoreMesh(core_axis_name="c", subcore_axis_name="s")
    NC = plsc.get_sparse_core_info().num_cores

    @functools.partial(
        pl.kernel,
        out_type=(
            jax.ShapeDtypeStruct((B, 2 * D), jnp.float32),
            jax.ShapeDtypeStruct((B, 2 * D), jnp.float32),
        ),
        mesh=mesh,
        scratch_types=[
            pltpu.VMEM((NBUF, IDX_CHUNK), jnp.int32),
            pltpu.VMEM((NBUF, IDX_CHUNK, 2 * D), jnp.float32),
            pltpu.SemaphoreType.DMA,
            pltpu.SemaphoreType.DMA,
        ],
    )
    def k(upairs, ipairs, uhi, ihi, upo, ipo, idx_v, rows_v, sem0, sem1):
        sems = (sem0, sem1)
        wid = lax.axis_index("s") * NC + lax.axis_index("c")
        units = []
        for t in range(2):
            tab = (upairs, ipairs)[t]
            ids = (uhi, ihi)[t]
            out = (upo, ipo)[t]
            for c in range(chunks):
                units.append((tab, ids, out, c))

        def issue(j, slot):
            tab, ids, out, c = units[j]
            row = wid * chunks + c
            pltpu.sync_copy(ids.at[row], idx_v.at[slot])
            return pltpu.async_copy(tab.at[idx_v.at[slot]], rows_v.at[slot],
                                    sems[slot])

        def retire(j, slot, cp):
            _, _, out, c = units[j]
            row = wid * chunks + c
            cp.wait()
            pltpu.sync_copy(rows_v.at[slot], out.at[pl.ds(row * IDX_CHUNK,
                                                          IDX_CHUNK)])

        inflight = []
        for j in range(len(units)):
            slot = j % NBUF
            if len(inflight) == NBUF:
                retire(j - NBUF, slot, inflight.pop(0))
            inflight.append(issue(j, slot))
        nu = len(units)
        for i, cp in enumerate(inflight):
            j = nu - len(inflight) + i
            retire(j, j % NBUF, cp)

    return k(user_pairs, item_pairs, uhi2, ihi2)


def _mlp_body(up_ref, ip_ref, upar_ref, ipar_ref,
              w0_ref, b0_ref, g0_ref, bt0_ref,
              w1_ref, b1_ref, g1_ref, bt1_ref,
              w2_ref, b2_ref, g2_ref, bt2_ref,
              wo_ref, bo_ref, out_ref):
    inv = 1.0 / math.sqrt(1.0 + 1e-5)
    upar = upar_ref[...]
    ipar = ipar_ref[...]
    ue = upar * up_ref[:, D:] + (1.0 - upar) * up_ref[:, :D]
    ie = ipar * ip_ref[:, D:] + (1.0 - ipar) * ip_ref[:, :D]
    x = (jnp.dot(ue, w0_ref[:D, :], preferred_element_type=jnp.float32)
         + jnp.dot(ie, w0_ref[D:, :], preferred_element_type=jnp.float32)
         + b0_ref[...])
    x = g0_ref[...] * (jnp.maximum(x, 0.0) * inv) + bt0_ref[...]
    x = jnp.dot(x, w1_ref[...], preferred_element_type=jnp.float32) + b1_ref[...]
    x = g1_ref[...] * (jnp.maximum(x, 0.0) * inv) + bt1_ref[...]
    x = jnp.dot(x, w2_ref[...], preferred_element_type=jnp.float32) + b2_ref[...]
    x = g2_ref[...] * (jnp.maximum(x, 0.0) * inv) + bt2_ref[...]
    o = jnp.dot(x, wo_ref[...], preferred_element_type=jnp.float32) + bo_ref[...]
    out_ref[...] = jax.nn.sigmoid(o)


def _mlp(up, ip, upar, ipar,
         W0, b0, g0, bt0, W1, b1, g1, bt1, W2, b2, g2, bt2, Wo, bo, block_m):
    B = up.shape[0]
    grid = (B // block_m,)

    def batch_spec(cols):
        return pl.BlockSpec((block_m, cols), lambda i: (i, 0))

    def full_spec(arr):
        return pl.BlockSpec(arr.shape, lambda i: (0,) * arr.ndim)

    row = lambda v: v.reshape(1, -1)
    args = (up, ip, upar, ipar,
            W0, row(b0), row(g0), row(bt0),
            W1, row(b1), row(g1), row(bt1),
            W2, row(b2), row(g2), row(bt2),
            Wo, row(bo))
    in_specs = ([batch_spec(2 * D), batch_spec(2 * D),
                 batch_spec(1), batch_spec(1)]
                + [full_spec(a) for a in args[4:]])
    return pl.pallas_call(
        _mlp_body,
        grid=grid,
        in_specs=in_specs,
        out_specs=pl.BlockSpec((block_m, 1), lambda i: (i, 0)),
        out_shape=jax.ShapeDtypeStruct((B, 1), jnp.float32),
    )(*args)


def kernel(user_ids, item_ids, user_table, item_table,
           W0, b0, gamma0, beta0,
           W1, b1, gamma1, beta1,
           W2, b2, gamma2, beta2,
           Wo, bo):
    B = user_ids.shape[0]
    n_users, d = user_table.shape
    n_items = item_table.shape[0]
    info = plsc.get_sparse_core_info()
    n_workers = info.num_cores * info.num_subcores
    chunks = B // (n_workers * IDX_CHUNK)

    upairs = user_table.reshape(n_users // 2, 2 * d)
    ipairs = item_table.reshape(n_items // 2, 2 * d)
    uid = user_ids.astype(jnp.int32)
    iid = item_ids.astype(jnp.int32)
    uhi2 = (uid >> 1).reshape(n_workers * chunks, IDX_CHUNK)
    ihi2 = (iid >> 1).reshape(n_workers * chunks, IDX_CHUNK)
    up, ipr = _sc_gather_pairs(upairs, ipairs, uhi2, ihi2, n_workers, chunks)
    upar = (uid & 1).astype(jnp.float32).reshape(B, 1)
    ipar = (iid & 1).astype(jnp.float32).reshape(B, 1)
    out = _mlp(up, ipr, upar, ipar,
               W0, b0, gamma0, beta0, W1, b1, gamma1, beta1,
               W2, b2, gamma2, beta2, Wo, bo, block_m=2048)
    return out.reshape(B)


# speedup vs baseline: 1.5113x; 1.5113x over previous
import functools
import math

import jax
import jax.numpy as jnp
from jax import lax
from jax.experimental import pallas as pl
from jax.experimental.pallas import tpu as pltpu
from jax.experimental.pallas import tpu_sc as plsc

D = 64
GRP = 8


def _sc_gather_pair(user_table, item_table, uid, iid):
    B = uid.shape[0]
    info = plsc.get_sparse_core_info()
    NC = info.num_cores
    NW = NC * info.num_subcores
    bpw = B // NW

    mesh = plsc.VectorSubcoreMesh(core_axis_name="c", subcore_axis_name="s")

    @functools.partial(
        pl.kernel,
        out_type=(
            jax.ShapeDtypeStruct((B, D), jnp.float32),
            jax.ShapeDtypeStruct((B, D), jnp.float32),
        ),
        mesh=mesh,
        scratch_types=[
            pltpu.VMEM((bpw,), jnp.int32),
            pltpu.VMEM((bpw,), jnp.int32),
            pltpu.VMEM((bpw, D), jnp.float32),
            pltpu.SemaphoreType.DMA,
        ],
    )
    def k(ut, it, uids, iids, ue_out, ie_out, uidx_v, iidx_v, rows_v, sem):
        wid = lax.axis_index("s") * NC + lax.axis_index("c")
        base = wid * bpw
        pltpu.sync_copy(uids.at[pl.ds(base, bpw)], uidx_v)
        pltpu.sync_copy(iids.at[pl.ds(base, bpw)], iidx_v)

        def gather_loop(idx_v, tab, rows_v):
            def body(g, _):
                r0 = g * 16
                vec = idx_v[pl.ds(r0, 16)]
                cps = [pltpu.async_copy(tab.at[pl.ds(vec[l], 1)],
                                        rows_v.at[pl.ds(r0 + l, 1)], sem)
                       for l in range(16)]
                for cp in cps:
                    cp.wait()
                return 0

            lax.fori_loop(0, bpw // 16, body, 0)

        gather_loop(uidx_v, ut, rows_v)
        pltpu.sync_copy(rows_v, ue_out.at[pl.ds(base, bpw)])
        gather_loop(iidx_v, it, rows_v)
        pltpu.sync_copy(rows_v, ie_out.at[pl.ds(base, bpw)])

    return k(user_table, item_table, uid, iid)


def _mlp_body(ue_ref, ie_ref, w0_ref, b0_ref, g0_ref, bt0_ref,
              w1_ref, b1_ref, g1_ref, bt1_ref,
              w2_ref, b2_ref, g2_ref, bt2_ref,
              wo_ref, bo_ref, out_ref):
    inv = 1.0 / math.sqrt(1.0 + 1e-5)
    x = (jnp.dot(ue_ref[...], w0_ref[:D, :], preferred_element_type=jnp.float32)
         + jnp.dot(ie_ref[...], w0_ref[D:, :], preferred_element_type=jnp.float32)
         + b0_ref[...])
    x = g0_ref[...] * (jnp.maximum(x, 0.0) * inv) + bt0_ref[...]
    x = jnp.dot(x, w1_ref[...], preferred_element_type=jnp.float32) + b1_ref[...]
    x = g1_ref[...] * (jnp.maximum(x, 0.0) * inv) + bt1_ref[...]
    x = jnp.dot(x, w2_ref[...], preferred_element_type=jnp.float32) + b2_ref[...]
    x = g2_ref[...] * (jnp.maximum(x, 0.0) * inv) + bt2_ref[...]
    o = jnp.dot(x, wo_ref[...], preferred_element_type=jnp.float32) + bo_ref[...]
    out_ref[...] = jax.nn.sigmoid(o)


def _mlp(ue, ie, W0, b0, g0, bt0, W1, b1, g1, bt1, W2, b2, g2, bt2, Wo, bo,
         block_m):
    B = ue.shape[0]
    grid = (B // block_m,)

    def batch_spec(cols):
        return pl.BlockSpec((block_m, cols), lambda i: (i, 0))

    def full_spec(arr):
        return pl.BlockSpec(arr.shape, lambda i: (0,) * arr.ndim)

    row = lambda v: v.reshape(1, -1)
    args = (ue, ie, W0, row(b0), row(g0), row(bt0),
            W1, row(b1), row(g1), row(bt1),
            W2, row(b2), row(g2), row(bt2),
            Wo, row(bo))
    in_specs = [batch_spec(D), batch_spec(D)] + [full_spec(a) for a in args[2:]]
    return pl.pallas_call(
        _mlp_body,
        grid=grid,
        in_specs=in_specs,
        out_specs=pl.BlockSpec((block_m, 1), lambda i: (i, 0)),
        out_shape=jax.ShapeDtypeStruct((B, 1), jnp.float32),
    )(*args)


def kernel(user_ids, item_ids, user_table, item_table,
           W0, b0, gamma0, beta0,
           W1, b1, gamma1, beta1,
           W2, b2, gamma2, beta2,
           Wo, bo):
    B = user_ids.shape[0]
    ue, ie = _sc_gather_pair(user_table, item_table,
                             user_ids.astype(jnp.int32),
                             item_ids.astype(jnp.int32))
    out = _mlp(ue, ie, W0, b0, gamma0, beta0, W1, b1, gamma1, beta1,
               W2, b2, gamma2, beta2, Wo, bo, block_m=2048)
    return out.reshape(B)
